# baseline (speedup 1.0000x reference)
import functools

import jax
import jax.numpy as jnp
from jax import lax
from jax.experimental import pallas as pl
from jax.experimental.pallas import tpu as pltpu
from jax.experimental.pallas import tpu_sc as plsc

NC = 2
NS = 16
CH = 128
NBUF = 2
DGRP = 8
EPT0 = 80
NHALF = 2
SLAB = 40


def _ceil_to(a, m):
    return (a + m - 1) // m * m


def _prep_body(nrows_x, erows_per_tile, erows_per_core,
               x2d, dst2d, emb, zeros, ones,
               embx_out, deg_out,
               idx_v, idxs, gbuf, o_v, gsem, dsem, deg_sh):
    c = lax.axis_index("c")
    s = lax.axis_index("s")
    wid = s * NC + c
    nw = NC * NS
    npad = deg_sh.shape[0]
    rows_per_tile = npad // NS
    base = s * rows_per_tile

    pltpu.sync_copy(zeros, gbuf)
    for k in range(rows_per_tile // CH):
        pltpu.sync_copy(gbuf, deg_sh.at[pl.ds(base + k * CH, CH)])
    pltpu.sync_copy(ones, o_v)

    def gather_row(row):
        pltpu.sync_copy(x2d.at[row], idx_v)
        pltpu.async_copy(emb.at[idx_v], gbuf, gsem).wait()
        pltpu.sync_copy(gbuf, embx_out.at[pl.ds(row * CH, CH)])

    lo = nrows_x // nw
    rem = nrows_x % nw
    for r in range(lo):
        gather_row(wid * lo + r)
    if rem:
        @pl.when(wid < rem)
        def _():
            gather_row(nw * lo + wid)

    plsc.subcore_barrier()

    erow0 = c * erows_per_core + s * erows_per_tile

    @pl.loop(0, erows_per_tile, step=DGRP)
    def _(i):
        for b in range(DGRP):
            pltpu.sync_copy(dst2d.at[erow0 + i + b], idxs.at[b])
            pltpu.async_copy(o_v, deg_sh.at[idxs.at[b]], dsem, add=True)
        for b in range(DGRP):
            pltpu.make_async_copy(zeros, o_v, dsem).wait()

    plsc.subcore_barrier()
    for k in range(rows_per_tile // CH):
        pltpu.sync_copy(deg_sh.at[pl.ds(base + k * CH, CH)], gbuf)
        pltpu.sync_copy(gbuf, deg_out.at[c, pl.ds(base + k * CH, CH)])


def _edge_body(src2d, dst2d, hv, zeros,
               acc_out,
               sslab, dslab, r0, r1,
               gs0, gs1, ss0, ss1, acc_sh):
    rows = (r0, r1)
    gsems = (gs0, gs1)
    ssems = (ss0, ss1)
    c = lax.axis_index("c")
    s = lax.axis_index("s")
    npad = acc_sh.shape[0]
    rows_per_tile = npad // NS
    base = s * rows_per_tile

    @pl.when(c == 0)
    def _():
        pltpu.sync_copy(zeros, rows[0])
        for k in range(rows_per_tile // CH):
            pltpu.sync_copy(rows[0], acc_sh.at[pl.ds(base + k * CH, CH)])
    plsc.subcore_barrier()

    def ring(erow0, nseg, seg):
        @pl.loop(0, nseg)
        def _(t):
            row0 = erow0 + t * seg
            sdst = sslab if seg == SLAB else sslab.at[pl.ds(0, seg)]
            ddst = dslab if seg == SLAB else dslab.at[pl.ds(0, seg)]
            pltpu.sync_copy(src2d.at[pl.ds(row0, seg)], sdst)
            pltpu.sync_copy(dst2d.at[pl.ds(row0, seg)], ddst)
            for b in range(NBUF):
                pltpu.async_copy(hv.at[sslab.at[b]], rows[b], gsems[b])

            @pl.loop(0, seg, step=NBUF)
            def _(i):
                for b in range(NBUF):
                    j = i + b
                    pltpu.make_async_copy(zeros, rows[b], gsems[b]).wait()
                    pltpu.async_copy(rows[b], acc_sh.at[dslab.at[j]],
                                     ssems[b], add=True)
                    pltpu.make_async_copy(zeros, rows[b], ssems[b]).wait()

                    @pl.when(j + NBUF < seg)
                    def _():
                        pltpu.async_copy(hv.at[sslab.at[j + NBUF]], rows[b],
                                         gsems[b])

    @pl.when(c == 0)
    def _():
        ring(s * EPT0, EPT0 // SLAB, SLAB)

    plsc.subcore_barrier()

    @pl.when(c == 0)
    def _():
        for k in range(rows_per_tile // CH):
            pltpu.sync_copy(acc_sh.at[pl.ds(base + k * CH, CH)], rows[k % NBUF])
            pltpu.sync_copy(rows[k % NBUF], acc_out.at[pl.ds(base + k * CH, CH)])


def _dinv_from(degp_ref):
    deg = degp_ref[0, :, 0:1] + degp_ref[1, :, 0:1] + 1.0
    return lax.rsqrt(deg)


def _tc1_body(embx_ref, w0_ref, degp_ref, hv0_ref):
    dinv = _dinv_from(degp_ref)
    hw = jnp.dot(embx_ref[...], w0_ref[...], preferred_element_type=jnp.float32)
    hv0_ref[...] = dinv * hw


def _ln_relu(t, g_ref, be_ref):
    mu = jnp.mean(t, axis=-1, keepdims=True)
    ctr = t - mu
    var = jnp.mean(ctr * ctr, axis=-1, keepdims=True)
    return jnp.maximum(ctr * lax.rsqrt(var + 1e-5) * g_ref[...] + be_ref[...], 0.0)


def _tc2_body(acca_ref, accb_ref, hv0_ref, degp_ref, embx_ref, wskip_ref,
              bskip_ref, w1_ref, b0_ref, g0_ref, be0_ref, hv1_ref):
    dinv = _dinv_from(degp_ref)
    t = dinv * (acca_ref[...] + accb_ref[...] + hv0_ref[...]) + b0_ref[...]
    h = _ln_relu(t, g0_ref, be0_ref)
    u = jnp.dot(embx_ref[...], wskip_ref[...],
                preferred_element_type=jnp.float32) + bskip_ref[...] + h
    hv1_ref[...] = dinv * jnp.dot(u, w1_ref[...],
                                  preferred_element_type=jnp.float32)


def _tc3_body(acca_ref, accb_ref, hv1_ref, degp_ref, b1_ref, g1_ref, be1_ref,
              lo_ref, hi_ref, ci_ref, h2_ref, g_ref):
    i = pl.program_id(0)
    rb = hv1_ref.shape[0]
    nb = lo_ref.shape[0]
    dinv = _dinv_from(degp_ref)
    t = dinv * (acca_ref[...] + accb_ref[...] + hv1_ref[...]) + b1_ref[...]
    h2 = _ln_relu(t, g1_ref, be1_ref)
    h2_ref[...] = h2
    gi = (lax.broadcasted_iota(jnp.int32, (nb, rb), 1) + i * rb).astype(jnp.float32)
    seg = jnp.where((gi >= lo_ref[...]) & (gi < hi_ref[...]), 1.0, 0.0)

    @pl.when(i == 0)
    def _():
        g_ref[...] = jnp.zeros_like(g_ref)

    g_ref[...] += jnp.dot(seg, h2, preferred_element_type=jnp.float32)

    @pl.when(i == pl.num_programs(0) - 1)
    def _():
        g_ref[...] = g_ref[...] * ci_ref[...]


def kernel(x, edge_index_x, ptr_x, y, edge_index_y, ptr_y, emb, w_skip,
           b_skip, w0, b0, w1, b1, g0, be0, g1, be1):
    n = x.shape[0]
    e = edge_index_x.shape[1]
    d = emb.shape[1]
    nb = ptr_x.shape[0] - 1
    f32 = jnp.float32

    npad = _ceil_to(n, NS * CH)
    nrows_x = npad // CH
    erows = NS * EPT0 * NHALF
    erows_per_tile = erows // (NC * NS)
    erows_per_core = erows // NC
    epad = erows * CH
    assert epad >= e and EPT0 % SLAB == 0

    x_pad = jnp.concatenate(
        [x.astype(jnp.int32), jnp.zeros((npad - n,), jnp.int32)]).reshape(nrows_x, CH)
    src = edge_index_x[0].astype(jnp.int32)
    dst = edge_index_x[1].astype(jnp.int32)
    src2d = jnp.concatenate(
        [src, jnp.zeros((epad - e,), jnp.int32)]).reshape(erows, CH)
    dst2d = jnp.concatenate(
        [dst, jnp.full((epad - e,), n, jnp.int32)]).reshape(erows, CH)
    zeros = jnp.zeros((CH, d), f32)
    ones = jnp.ones((CH, d), f32)

    mesh = plsc.VectorSubcoreMesh(core_axis_name="c", subcore_axis_name="s")

    prep = pl.kernel(
        functools.partial(_prep_body, nrows_x, erows_per_tile, erows_per_core),
        out_type=(
            jax.ShapeDtypeStruct((npad, d), f32),
            jax.ShapeDtypeStruct((NC, npad, d), f32),
        ),
        mesh=mesh,
        scratch_types=[
            pltpu.VMEM((CH,), jnp.int32),
            pltpu.VMEM((DGRP, CH), jnp.int32),
            pltpu.VMEM((CH, d), f32),
            pltpu.VMEM((CH, d), f32),
            pltpu.SemaphoreType.DMA,
            pltpu.SemaphoreType.DMA,
            pltpu.VMEM_SHARED((npad, d), f32),
        ],
    )

    edge_pass = pl.kernel(
        _edge_body,
        out_type=jax.ShapeDtypeStruct((npad, d), f32),
        mesh=mesh,
        scratch_types=(
            [pltpu.VMEM((SLAB, CH), jnp.int32)] * 2
            + [pltpu.VMEM((CH, d), f32)] * NBUF
            + [pltpu.SemaphoreType.DMA] * (2 * NBUF)
            + [pltpu.VMEM_SHARED((npad, d), f32)]
        ),
    )

    embx, degp = prep(x_pad, dst2d, emb, zeros, ones)
    hrows = erows // NHALF
    src_h = [src2d[k * hrows:(k + 1) * hrows] for k in range(NHALF)]
    dst_h = [dst2d[k * hrows:(k + 1) * hrows] for k in range(NHALF)]

    rb = 1024
    grid = npad // rb
    row_spec = pl.BlockSpec((rb, d), lambda i: (i, 0))
    acc_spec = pl.BlockSpec((NC, rb, d), lambda i: (0, i, 0))
    mat_spec = pl.BlockSpec((d, d), lambda i: (0, 0))
    vec_spec = pl.BlockSpec((1, d), lambda i: (0, 0))

    hv0 = pl.pallas_call(
        _tc1_body,
        grid=(grid,),
        in_specs=[row_spec, mat_spec, acc_spec],
        out_specs=row_spec,
        out_shape=jax.ShapeDtypeStruct((npad, d), f32),
    )(embx, w0, degp)

    acc0a = edge_pass(src_h[0], dst_h[0], hv0, zeros)
    acc0b = edge_pass(src_h[1], dst_h[1], hv0, zeros)

    b0r = b0.reshape(1, d)
    g0r = g0.reshape(1, d)
    be0r = be0.reshape(1, d)
    bskipr = b_skip.reshape(1, d)
    hv1 = pl.pallas_call(
        _tc2_body,
        grid=(grid,),
        in_specs=[row_spec, row_spec, row_spec, acc_spec, row_spec, mat_spec,
                  vec_spec, mat_spec, vec_spec, vec_spec, vec_spec],
        out_specs=row_spec,
        out_shape=jax.ShapeDtypeStruct((npad, d), f32),
    )(acc0a, acc0b, hv0, degp, embx, w_skip, bskipr, w1, b0r, g0r, be0r)

    acc1a = edge_pass(src_h[0], dst_h[0], hv1, zeros)
    acc1b = edge_pass(src_h[1], dst_h[1], hv1, zeros)

    lo = ptr_x[:-1].astype(f32).reshape(nb, 1)
    hi = ptr_x[1:].astype(f32).reshape(nb, 1)
    ci = 1.0 / jnp.maximum(hi - lo, 1.0)
    b1r = b1.reshape(1, d)
    g1r = g1.reshape(1, d)
    be1r = be1.reshape(1, d)
    seg_spec = pl.BlockSpec((nb, 1), lambda i: (0, 0))
    h2_pad, g_x = pl.pallas_call(
        _tc3_body,
        grid=(grid,),
        in_specs=[row_spec, row_spec, row_spec, acc_spec, vec_spec, vec_spec,
                  vec_spec, seg_spec, seg_spec, seg_spec],
        out_specs=[row_spec, pl.BlockSpec((nb, d), lambda i: (0, 0))],
        out_shape=[jax.ShapeDtypeStruct((npad, d), f32),
                   jax.ShapeDtypeStruct((nb, d), f32)],
    )(acc1a, acc1b, hv1, degp, b1r, g1r, be1r, lo, hi, ci)

    return (h2_pad[:n], g_x)


# speedup vs baseline: 1.4631x; 1.4631x over previous
import functools

import jax
import jax.numpy as jnp
from jax import lax
from jax.experimental import pallas as pl
from jax.experimental.pallas import tpu as pltpu
from jax.experimental.pallas import tpu_sc as plsc

NC = 2
NS = 16
CH = 128
NBUF = 2
DGRP = 8
EPT0 = 144
EPT1 = 16
SLAB = 48


def _ceil_to(a, m):
    return (a + m - 1) // m * m


def _prep_body(nrows_x, erows_per_tile, erows_per_core,
               x2d, dst2d, emb, zeros, ones,
               embx_out, deg_out,
               idx_v, idxs, gbuf, o_v, gsem, dsem, deg_sh):
    c = lax.axis_index("c")
    s = lax.axis_index("s")
    wid = s * NC + c
    nw = NC * NS
    npad = deg_sh.shape[0]
    rows_per_tile = npad // NS
    base = s * rows_per_tile

    pltpu.sync_copy(zeros, gbuf)
    for k in range(rows_per_tile // CH):
        pltpu.sync_copy(gbuf, deg_sh.at[pl.ds(base + k * CH, CH)])
    pltpu.sync_copy(ones, o_v)

    def gather_row(row):
        pltpu.sync_copy(x2d.at[row], idx_v)
        pltpu.async_copy(emb.at[idx_v], gbuf, gsem).wait()
        pltpu.sync_copy(gbuf, embx_out.at[pl.ds(row * CH, CH)])

    lo = nrows_x // nw
    rem = nrows_x % nw
    for r in range(lo):
        gather_row(wid * lo + r)
    if rem:
        @pl.when(wid < rem)
        def _():
            gather_row(nw * lo + wid)

    plsc.subcore_barrier()

    erow0 = c * erows_per_core + s * erows_per_tile

    @pl.loop(0, erows_per_tile, step=DGRP)
    def _(i):
        for b in range(DGRP):
            pltpu.sync_copy(dst2d.at[erow0 + i + b], idxs.at[b])
            pltpu.async_copy(o_v, deg_sh.at[idxs.at[b]], dsem, add=True)
        for b in range(DGRP):
            pltpu.make_async_copy(zeros, o_v, dsem).wait()

    plsc.subcore_barrier()
    for k in range(rows_per_tile // CH):
        pltpu.sync_copy(deg_sh.at[pl.ds(base + k * CH, CH)], gbuf)
        pltpu.sync_copy(gbuf, deg_out.at[c, pl.ds(base + k * CH, CH)])


def _edge_body(src2d, dst2d, hv, zeros,
               acc_out,
               sslab, dslab, r0, r1,
               gs0, gs1, ss0, ss1, acc_sh):
    rows = (r0, r1)
    gsems = (gs0, gs1)
    ssems = (ss0, ss1)
    c = lax.axis_index("c")
    s = lax.axis_index("s")
    npad = acc_sh.shape[0]
    rows_per_tile = npad // NS
    base = s * rows_per_tile

    pltpu.sync_copy(zeros, rows[0])
    for k in range(rows_per_tile // CH):
        pltpu.sync_copy(rows[0], acc_sh.at[pl.ds(base + k * CH, CH)])
    plsc.subcore_barrier()

    def ring(erow0, nseg, seg):
        @pl.loop(0, nseg)
        def _(t):
            row0 = erow0 + t * seg
            sdst = sslab if seg == SLAB else sslab.at[pl.ds(0, seg)]
            ddst = dslab if seg == SLAB else dslab.at[pl.ds(0, seg)]
            pltpu.sync_copy(src2d.at[pl.ds(row0, seg)], sdst)
            pltpu.sync_copy(dst2d.at[pl.ds(row0, seg)], ddst)
            for b in range(NBUF):
                pltpu.async_copy(hv.at[sslab.at[b]], rows[b], gsems[b])

            @pl.loop(0, seg, step=NBUF)
            def _(i):
                for b in range(NBUF):
                    j = i + b
                    pltpu.make_async_copy(zeros, rows[b], gsems[b]).wait()
                    pltpu.async_copy(rows[b], acc_sh.at[dslab.at[j]],
                                     ssems[b], add=True)
                    pltpu.make_async_copy(zeros, rows[b], ssems[b]).wait()

                    @pl.when(j + NBUF < seg)
                    def _():
                        pltpu.async_copy(hv.at[sslab.at[j + NBUF]], rows[b],
                                         gsems[b])

    @pl.when(c == 0)
    def _():
        ring(s * EPT0, EPT0 // SLAB, SLAB)

    @pl.when(c == 1)
    def _():
        ring(NS * EPT0 + s * EPT1, 1, EPT1)

    plsc.subcore_barrier()
    for k in range(rows_per_tile // CH):
        pltpu.sync_copy(acc_sh.at[pl.ds(base + k * CH, CH)], rows[k % NBUF])
        pltpu.sync_copy(rows[k % NBUF], acc_out.at[c, pl.ds(base + k * CH, CH)])


def _dinv_from(degp_ref):
    deg = degp_ref[0, :, 0:1] + degp_ref[1, :, 0:1] + 1.0
    return lax.rsqrt(deg)


def _tc1_body(embx_ref, w0_ref, degp_ref, hv0_ref):
    dinv = _dinv_from(degp_ref)
    hw = jnp.dot(embx_ref[...], w0_ref[...], preferred_element_type=jnp.float32)
    hv0_ref[...] = dinv * hw


def _ln_relu(t, g_ref, be_ref):
    mu = jnp.mean(t, axis=-1, keepdims=True)
    ctr = t - mu
    var = jnp.mean(ctr * ctr, axis=-1, keepdims=True)
    return jnp.maximum(ctr * lax.rsqrt(var + 1e-5) * g_ref[...] + be_ref[...], 0.0)


def _tc2_body(acc_ref, hv0_ref, degp_ref, embx_ref, wskip_ref, bskip_ref,
              w1_ref, b0_ref, g0_ref, be0_ref, hv1_ref):
    dinv = _dinv_from(degp_ref)
    t = dinv * (acc_ref[0] + acc_ref[1] + hv0_ref[...]) + b0_ref[...]
    h = _ln_relu(t, g0_ref, be0_ref)
    u = jnp.dot(embx_ref[...], wskip_ref[...],
                preferred_element_type=jnp.float32) + bskip_ref[...] + h
    hv1_ref[...] = dinv * jnp.dot(u, w1_ref[...],
                                  preferred_element_type=jnp.float32)


def _tc3_body(acc_ref, hv1_ref, degp_ref, b1_ref, g1_ref, be1_ref,
              lo_ref, hi_ref, ci_ref, h2_ref, g_ref):
    i = pl.program_id(0)
    rb = hv1_ref.shape[0]
    nb = lo_ref.shape[0]
    dinv = _dinv_from(degp_ref)
    t = dinv * (acc_ref[0] + acc_ref[1] + hv1_ref[...]) + b1_ref[...]
    h2 = _ln_relu(t, g1_ref, be1_ref)
    h2_ref[...] = h2
    gi = (lax.broadcasted_iota(jnp.int32, (nb, rb), 1) + i * rb).astype(jnp.float32)
    seg = jnp.where((gi >= lo_ref[...]) & (gi < hi_ref[...]), 1.0, 0.0)

    @pl.when(i == 0)
    def _():
        g_ref[...] = jnp.zeros_like(g_ref)

    g_ref[...] += jnp.dot(seg, h2, preferred_element_type=jnp.float32)

    @pl.when(i == pl.num_programs(0) - 1)
    def _():
        g_ref[...] = g_ref[...] * ci_ref[...]


def kernel(x, edge_index_x, ptr_x, y, edge_index_y, ptr_y, emb, w_skip,
           b_skip, w0, b0, w1, b1, g0, be0, g1, be1):
    n = x.shape[0]
    e = edge_index_x.shape[1]
    d = emb.shape[1]
    nb = ptr_x.shape[0] - 1
    f32 = jnp.float32

    npad = _ceil_to(n, NS * CH)
    nrows_x = npad // CH
    erows = NS * (EPT0 + EPT1)
    erows_per_tile = erows // (NC * NS)
    erows_per_core = erows // NC
    epad = erows * CH
    assert epad >= e and EPT0 % SLAB == 0 and EPT1 % NBUF == 0

    x_pad = jnp.concatenate(
        [x.astype(jnp.int32), jnp.zeros((npad - n,), jnp.int32)]).reshape(nrows_x, CH)
    src = edge_index_x[0].astype(jnp.int32)
    dst = edge_index_x[1].astype(jnp.int32)
    src2d = jnp.concatenate(
        [src, jnp.zeros((epad - e,), jnp.int32)]).reshape(erows, CH)
    dst2d = jnp.concatenate(
        [dst, jnp.full((epad - e,), n, jnp.int32)]).reshape(erows, CH)
    zeros = jnp.zeros((CH, d), f32)
    ones = jnp.ones((CH, d), f32)

    mesh = plsc.VectorSubcoreMesh(core_axis_name="c", subcore_axis_name="s")

    prep = pl.kernel(
        functools.partial(_prep_body, nrows_x, erows_per_tile, erows_per_core),
        out_type=(
            jax.ShapeDtypeStruct((npad, d), f32),
            jax.ShapeDtypeStruct((NC, npad, d), f32),
        ),
        mesh=mesh,
        scratch_types=[
            pltpu.VMEM((CH,), jnp.int32),
            pltpu.VMEM((DGRP, CH), jnp.int32),
            pltpu.VMEM((CH, d), f32),
            pltpu.VMEM((CH, d), f32),
            pltpu.SemaphoreType.DMA,
            pltpu.SemaphoreType.DMA,
            pltpu.VMEM_SHARED((npad, d), f32),
        ],
    )

    edge_pass = pl.kernel(
        _edge_body,
        out_type=jax.ShapeDtypeStruct((NC, npad, d), f32),
        mesh=mesh,
        scratch_types=(
            [pltpu.VMEM((SLAB, CH), jnp.int32)] * 2
            + [pltpu.VMEM((CH, d), f32)] * NBUF
            + [pltpu.SemaphoreType.DMA] * (2 * NBUF)
            + [pltpu.VMEM_SHARED((npad, d), f32)]
        ),
    )

    embx, degp = prep(x_pad, dst2d, emb, zeros, ones)

    rb = 1024
    grid = npad // rb
    row_spec = pl.BlockSpec((rb, d), lambda i: (i, 0))
    acc_spec = pl.BlockSpec((NC, rb, d), lambda i: (0, i, 0))
    mat_spec = pl.BlockSpec((d, d), lambda i: (0, 0))
    vec_spec = pl.BlockSpec((1, d), lambda i: (0, 0))

    hv0 = pl.pallas_call(
        _tc1_body,
        grid=(grid,),
        in_specs=[row_spec, mat_spec, acc_spec],
        out_specs=row_spec,
        out_shape=jax.ShapeDtypeStruct((npad, d), f32),
    )(embx, w0, degp)

    acc0 = edge_pass(src2d, dst2d, hv0, zeros)

    b0r = b0.reshape(1, d)
    g0r = g0.reshape(1, d)
    be0r = be0.reshape(1, d)
    bskipr = b_skip.reshape(1, d)
    hv1 = pl.pallas_call(
        _tc2_body,
        grid=(grid,),
        in_specs=[acc_spec, row_spec, acc_spec, row_spec, mat_spec, vec_spec,
                  mat_spec, vec_spec, vec_spec, vec_spec],
        out_specs=row_spec,
        out_shape=jax.ShapeDtypeStruct((npad, d), f32),
    )(acc0, hv0, degp, embx, w_skip, bskipr, w1, b0r, g0r, be0r)

    acc1 = edge_pass(src2d, dst2d, hv1, zeros)

    lo = ptr_x[:-1].astype(f32).reshape(nb, 1)
    hi = ptr_x[1:].astype(f32).reshape(nb, 1)
    ci = 1.0 / jnp.maximum(hi - lo, 1.0)
    b1r = b1.reshape(1, d)
    g1r = g1.reshape(1, d)
    be1r = be1.reshape(1, d)
    seg_spec = pl.BlockSpec((nb, 1), lambda i: (0, 0))
    h2_pad, g_x = pl.pallas_call(
        _tc3_body,
        grid=(grid,),
        in_specs=[acc_spec, row_spec, acc_spec, vec_spec, vec_spec, vec_spec,
                  seg_spec, seg_spec, seg_spec],
        out_specs=[row_spec, pl.BlockSpec((nb, d), lambda i: (0, 0))],
        out_shape=[jax.ShapeDtypeStruct((npad, d), f32),
                   jax.ShapeDtypeStruct((nb, d), f32)],
    )(acc1, hv1, degp, b1r, g1r, be1r, lo, hi, ci)

    return (h2_pad[:n], g_x)


# speedup vs baseline: 1.4655x; 1.0016x over previous
import functools

import jax
import jax.numpy as jnp
from jax import lax
from jax.experimental import pallas as pl
from jax.experimental.pallas import tpu as pltpu
from jax.experimental.pallas import tpu_sc as plsc

NC = 2
NS = 16
CH = 128
NBUF = 2
DGRP = 16
EPT0 = 144
EPT1 = 16
SLAB = 48


def _ceil_to(a, m):
    return (a + m - 1) // m * m


def _prep_body(nrows_x, erows_per_tile, erows_per_core,
               x2d, dst2d, emb, zeros, ones,
               embx_out, deg_out,
               idx_v, idxs, gbuf, o_v, gsem, dsem, deg_sh):
    c = lax.axis_index("c")
    s = lax.axis_index("s")
    wid = s * NC + c
    nw = NC * NS
    npad = deg_sh.shape[0]
    rows_per_tile = npad // NS
    base = s * rows_per_tile

    pltpu.sync_copy(zeros, gbuf)
    for k in range(rows_per_tile // CH):
        pltpu.sync_copy(gbuf, deg_sh.at[pl.ds(base + k * CH, CH)])
    pltpu.sync_copy(ones, o_v)

    def gather_row(row):
        pltpu.sync_copy(x2d.at[row], idx_v)
        pltpu.async_copy(emb.at[idx_v], gbuf, gsem).wait()
        pltpu.sync_copy(gbuf, embx_out.at[pl.ds(row * CH, CH)])

    lo = nrows_x // nw
    rem = nrows_x % nw
    for r in range(lo):
        gather_row(wid * lo + r)
    if rem:
        @pl.when(wid < rem)
        def _():
            gather_row(nw * lo + wid)

    plsc.subcore_barrier()

    erow0 = c * erows_per_core + s * erows_per_tile

    @pl.loop(0, erows_per_tile, step=DGRP)
    def _(i):
        for b in range(DGRP):
            pltpu.sync_copy(dst2d.at[erow0 + i + b], idxs.at[b])
            pltpu.async_copy(o_v, deg_sh.at[idxs.at[b]], dsem, add=True)
        for b in range(DGRP):
            pltpu.make_async_copy(zeros, o_v, dsem).wait()

    plsc.subcore_barrier()
    for k in range(rows_per_tile // CH):
        pltpu.sync_copy(deg_sh.at[pl.ds(base + k * CH, CH)], gbuf)
        pltpu.sync_copy(gbuf, deg_out.at[c, pl.ds(base + k * CH, CH)])


def _edge_body(src2d, dst2d, hv, zeros,
               acc_out,
               sslab, dslab, r0, r1,
               gs0, gs1, ss0, ss1, acc_sh):
    rows = (r0, r1)
    gsems = (gs0, gs1)
    ssems = (ss0, ss1)
    c = lax.axis_index("c")
    s = lax.axis_index("s")
    npad = acc_sh.shape[0]
    rows_per_tile = npad // NS
    base = s * rows_per_tile

    pltpu.sync_copy(zeros, rows[0])
    for k in range(rows_per_tile // CH):
        pltpu.sync_copy(rows[0], acc_sh.at[pl.ds(base + k * CH, CH)])
    plsc.subcore_barrier()

    def ring(erow0, nseg, seg):
        @pl.loop(0, nseg)
        def _(t):
            row0 = erow0 + t * seg
            sdst = sslab if seg == SLAB else sslab.at[pl.ds(0, seg)]
            ddst = dslab if seg == SLAB else dslab.at[pl.ds(0, seg)]
            pltpu.sync_copy(src2d.at[pl.ds(row0, seg)], sdst)
            pltpu.sync_copy(dst2d.at[pl.ds(row0, seg)], ddst)
            for b in range(NBUF):
                pltpu.async_copy(hv.at[sslab.at[b]], rows[b], gsems[b])

            @pl.loop(0, seg, step=NBUF)
            def _(i):
                for b in range(NBUF):
                    j = i + b
                    pltpu.make_async_copy(zeros, rows[b], gsems[b]).wait()
                    pltpu.async_copy(rows[b], acc_sh.at[dslab.at[j]],
                                     ssems[b], add=True)
                    pltpu.make_async_copy(zeros, rows[b], ssems[b]).wait()

                    @pl.when(j + NBUF < seg)
                    def _():
                        pltpu.async_copy(hv.at[sslab.at[j + NBUF]], rows[b],
                                         gsems[b])

    @pl.when(c == 0)
    def _():
        ring(s * EPT0, EPT0 // SLAB, SLAB)

    @pl.when(c == 1)
    def _():
        ring(NS * EPT0 + s * EPT1, 1, EPT1)

    plsc.subcore_barrier()
    for k in range(rows_per_tile // CH):
        pltpu.sync_copy(acc_sh.at[pl.ds(base + k * CH, CH)], rows[k % NBUF])
        pltpu.sync_copy(rows[k % NBUF], acc_out.at[c, pl.ds(base + k * CH, CH)])


def _dinv_from(degp_ref):
    deg = degp_ref[0, :, 0:1] + degp_ref[1, :, 0:1] + 1.0
    return lax.rsqrt(deg)


def _tc1_body(embx_ref, w0_ref, degp_ref, hv0_ref):
    dinv = _dinv_from(degp_ref)
    hw = jnp.dot(embx_ref[...], w0_ref[...], preferred_element_type=jnp.float32)
    hv0_ref[...] = dinv * hw


def _ln_relu(t, g_ref, be_ref):
    mu = jnp.mean(t, axis=-1, keepdims=True)
    ctr = t - mu
    var = jnp.mean(ctr * ctr, axis=-1, keepdims=True)
    return jnp.maximum(ctr * lax.rsqrt(var + 1e-5) * g_ref[...] + be_ref[...], 0.0)


def _tc2_body(acc_ref, hv0_ref, degp_ref, embx_ref, wskip_ref, bskip_ref,
              w1_ref, b0_ref, g0_ref, be0_ref, hv1_ref):
    dinv = _dinv_from(degp_ref)
    t = dinv * (acc_ref[0] + acc_ref[1] + hv0_ref[...]) + b0_ref[...]
    h = _ln_relu(t, g0_ref, be0_ref)
    u = jnp.dot(embx_ref[...], wskip_ref[...],
                preferred_element_type=jnp.float32) + bskip_ref[...] + h
    hv1_ref[...] = dinv * jnp.dot(u, w1_ref[...],
                                  preferred_element_type=jnp.float32)


def _tc3_body(acc_ref, hv1_ref, degp_ref, b1_ref, g1_ref, be1_ref,
              lo_ref, hi_ref, ci_ref, h2_ref, g_ref):
    i = pl.program_id(0)
    rb = hv1_ref.shape[0]
    nb = lo_ref.shape[0]
    dinv = _dinv_from(degp_ref)
    t = dinv * (acc_ref[0] + acc_ref[1] + hv1_ref[...]) + b1_ref[...]
    h2 = _ln_relu(t, g1_ref, be1_ref)
    h2_ref[...] = h2
    gi = (lax.broadcasted_iota(jnp.int32, (nb, rb), 1) + i * rb).astype(jnp.float32)
    seg = jnp.where((gi >= lo_ref[...]) & (gi < hi_ref[...]), 1.0, 0.0)

    @pl.when(i == 0)
    def _():
        g_ref[...] = jnp.zeros_like(g_ref)

    g_ref[...] += jnp.dot(seg, h2, preferred_element_type=jnp.float32)

    @pl.when(i == pl.num_programs(0) - 1)
    def _():
        g_ref[...] = g_ref[...] * ci_ref[...]


def kernel(x, edge_index_x, ptr_x, y, edge_index_y, ptr_y, emb, w_skip,
           b_skip, w0, b0, w1, b1, g0, be0, g1, be1):
    n = x.shape[0]
    e = edge_index_x.shape[1]
    d = emb.shape[1]
    nb = ptr_x.shape[0] - 1
    f32 = jnp.float32

    npad = _ceil_to(n, NS * CH)
    nrows_x = npad // CH
    erows = NS * (EPT0 + EPT1)
    erows_per_tile = erows // (NC * NS)
    erows_per_core = erows // NC
    epad = erows * CH
    assert epad >= e and EPT0 % SLAB == 0 and EPT1 % NBUF == 0

    x_pad = jnp.concatenate(
        [x.astype(jnp.int32), jnp.zeros((npad - n,), jnp.int32)]).reshape(nrows_x, CH)
    src = edge_index_x[0].astype(jnp.int32)
    dst = edge_index_x[1].astype(jnp.int32)
    src2d = jnp.concatenate(
        [src, jnp.zeros((epad - e,), jnp.int32)]).reshape(erows, CH)
    dst2d = jnp.concatenate(
        [dst, jnp.full((epad - e,), n, jnp.int32)]).reshape(erows, CH)
    zeros = jnp.zeros((CH, d), f32)
    ones = jnp.ones((CH, d), f32)

    mesh = plsc.VectorSubcoreMesh(core_axis_name="c", subcore_axis_name="s")

    prep = pl.kernel(
        functools.partial(_prep_body, nrows_x, erows_per_tile, erows_per_core),
        out_type=(
            jax.ShapeDtypeStruct((npad, d), f32),
            jax.ShapeDtypeStruct((NC, npad, d), f32),
        ),
        mesh=mesh,
        scratch_types=[
            pltpu.VMEM((CH,), jnp.int32),
            pltpu.VMEM((DGRP, CH), jnp.int32),
            pltpu.VMEM((CH, d), f32),
            pltpu.VMEM((CH, d), f32),
            pltpu.SemaphoreType.DMA,
            pltpu.SemaphoreType.DMA,
            pltpu.VMEM_SHARED((npad, d), f32),
        ],
    )

    edge_pass = pl.kernel(
        _edge_body,
        out_type=jax.ShapeDtypeStruct((NC, npad, d), f32),
        mesh=mesh,
        scratch_types=(
            [pltpu.VMEM((SLAB, CH), jnp.int32)] * 2
            + [pltpu.VMEM((CH, d), f32)] * NBUF
            + [pltpu.SemaphoreType.DMA] * (2 * NBUF)
            + [pltpu.VMEM_SHARED((npad, d), f32)]
        ),
    )

    embx, degp = prep(x_pad, dst2d, emb, zeros, ones)

    rb = 1024
    grid = npad // rb
    row_spec = pl.BlockSpec((rb, d), lambda i: (i, 0))
    acc_spec = pl.BlockSpec((NC, rb, d), lambda i: (0, i, 0))
    mat_spec = pl.BlockSpec((d, d), lambda i: (0, 0))
    vec_spec = pl.BlockSpec((1, d), lambda i: (0, 0))

    hv0 = pl.pallas_call(
        _tc1_body,
        grid=(grid,),
        in_specs=[row_spec, mat_spec, acc_spec],
        out_specs=row_spec,
        out_shape=jax.ShapeDtypeStruct((npad, d), f32),
    )(embx, w0, degp)

    acc0 = edge_pass(src2d, dst2d, hv0, zeros)

    b0r = b0.reshape(1, d)
    g0r = g0.reshape(1, d)
    be0r = be0.reshape(1, d)
    bskipr = b_skip.reshape(1, d)
    hv1 = pl.pallas_call(
        _tc2_body,
        grid=(grid,),
        in_specs=[acc_spec, row_spec, acc_spec, row_spec, mat_spec, vec_spec,
                  mat_spec, vec_spec, vec_spec, vec_spec],
        out_specs=row_spec,
        out_shape=jax.ShapeDtypeStruct((npad, d), f32),
    )(acc0, hv0, degp, embx, w_skip, bskipr, w1, b0r, g0r, be0r)

    acc1 = edge_pass(src2d, dst2d, hv1, zeros)

    lo = ptr_x[:-1].astype(f32).reshape(nb, 1)
    hi = ptr_x[1:].astype(f32).reshape(nb, 1)
    ci = 1.0 / jnp.maximum(hi - lo, 1.0)
    b1r = b1.reshape(1, d)
    g1r = g1.reshape(1, d)
    be1r = be1.reshape(1, d)
    seg_spec = pl.BlockSpec((nb, 1), lambda i: (0, 0))
    h2_pad, g_x = pl.pallas_call(
        _tc3_body,
        grid=(grid,),
        in_specs=[acc_spec, row_spec, acc_spec, vec_spec, vec_spec, vec_spec,
                  seg_spec, seg_spec, seg_spec],
        out_specs=[row_spec, pl.BlockSpec((nb, d), lambda i: (0, 0))],
        out_shape=[jax.ShapeDtypeStruct((npad, d), f32),
                   jax.ShapeDtypeStruct((nb, d), f32)],
    )(acc1, hv1, degp, b1r, g1r, be1r, lo, hi, ci)

    return (h2_pad[:n], g_x)
